# baseline (speedup 1.0000x reference)
import functools

import jax
import jax.numpy as jnp
from jax.experimental import pallas as pl


def _taps_body(x_ref, w_ref, b_ref, o_ref, *, shifts, cols, blk, co, act, nsp):
    base = pl.program_id(0) * blk
    accs = [jnp.zeros((blk, co), jnp.float32) for _ in range(nsp)]
    for t, (sh, col) in enumerate(zip(shifts, cols)):
        al, rem = sh - sh % 8, sh % 8
        xs = x_ref[pl.ds(base + al, blk + 8), :][rem:rem + blk, :]
        accs[col] = accs[col] + jnp.dot(xs, w_ref[t],
                                        preferred_element_type=jnp.float32)
    y = jnp.concatenate(accs, axis=1) if nsp > 1 else accs[0]
    y = y + b_ref[...]
    if act == "relu":
        y = jnp.maximum(y, 0.0)
    elif act == "tanh":
        y = jnp.tanh(y)
    o_ref[...] = y


def _taps_matmul(x2, w, b, shifts, cols, n_valid, blk, act):
    t, k, co = w.shape
    ct = b.shape[1]
    nsp = ct // co
    grid = -(-n_valid // blk)
    n_out = grid * blk
    max_end = (grid - 1) * blk + max(shifts) + blk + 8
    x2p = jnp.pad(x2, ((0, max_end - x2.shape[0]), (0, 0)))
    body = functools.partial(_taps_body, shifts=tuple(shifts), cols=tuple(cols),
                             blk=blk, co=co, act=act, nsp=nsp)
    return pl.pallas_call(
        body,
        grid=(grid,),
        in_specs=[
            pl.BlockSpec(x2p.shape, lambda i: (0, 0)),
            pl.BlockSpec(w.shape, lambda i: (0, 0, 0)),
            pl.BlockSpec(b.shape, lambda i: (0, 0)),
        ],
        out_specs=pl.BlockSpec((blk, ct), lambda i: (i, 0)),
        out_shape=jax.ShapeDtypeStruct((n_out, ct), jnp.float32),
    )(x2p, w, b)


def _conv_s1(a, w, bias, act, blk):
    bsz, h, wd, cin = a.shape
    cout = w.shape[0]
    xp = jnp.pad(a, ((0, 0), (2, 2), (2, 2), (0, 0)))
    hp = h + 4
    x2 = xp.reshape(bsz * hp * hp, cin)
    front = hp + 1
    x2 = jnp.pad(x2, ((front, 0), (0, 0)))
    wt = w.transpose(2, 3, 1, 0)
    taps = [wt[kh, kw] for kh in range(3) for kw in range(3)]
    shifts = [front + (kh - 1) * hp + (kw - 1)
              for kh in range(3) for kw in range(3)]
    ws = jnp.stack(taps)
    n = bsz * hp * hp
    y = _taps_matmul(x2, ws, bias[None, :], shifts, [0] * 9, n, blk, act)
    return y[:n].reshape(bsz, hp, hp, cout)[:, 2:2 + h, 2:2 + wd, :]


def _conv_t(a, w, bias, act):
    bsz, h, wd, cin = a.shape
    cout = w.shape[1]
    xp = jnp.pad(a, ((0, 0), (1, 1), (1, 1), (0, 0)))
    hp = h + 2
    x2 = xp.reshape(bsz * hp * hp, cin)
    taps, shifts, cols = [], [], []
    for r in (0, 1):
        for s in (0, 1):
            for di in (0, 1):
                for dj in (0, 1):
                    taps.append(w[:, :, 3 - 2 * di - r, 3 - 2 * dj - s])
                    shifts.append((r + di) * hp + (s + dj))
                    cols.append(2 * r + s)
    ws = jnp.stack(taps)
    btile = jnp.tile(bias[None, :], (1, 4))
    n = bsz * hp * hp
    y = _taps_matmul(x2, ws, btile, shifts, cols, n, 512, act)
    y = y[:n].reshape(bsz, hp, hp, 4, cout)[:, :h, :wd]
    y = y.reshape(bsz, h, wd, 2, 2, cout).transpose(0, 1, 3, 2, 4, 5)
    return y.reshape(bsz, 2 * h, 2 * wd, cout)


_VQ_BLK = 784


def _vq_body(x_ref, cb_ref, cb2_ref, q_ref, idx_ref, loss_ref, *, nq, blk, k):
    r = x_ref[...]
    qsum = jnp.zeros_like(r)
    lanef = jax.lax.broadcasted_iota(jnp.int32, (blk, k), 1).astype(jnp.float32)
    losses = []
    for q in range(nq):
        cb = cb_ref[q]
        r2 = jnp.sum(r * r, axis=1, keepdims=True)
        mm = jax.lax.dot_general(r, cb, (((1,), (1,)), ((), ())),
                                 preferred_element_type=jnp.float32)
        d = (r2 - 2.0 * mm) + cb2_ref[q]
        m = jnp.min(d, axis=1, keepdims=True)
        idxf = jnp.min(jnp.where(d <= m, lanef, jnp.float32(k)),
                       axis=1, keepdims=True)
        oh = (lanef == idxf).astype(jnp.float32)
        qv = jnp.dot(oh, cb, precision=jax.lax.Precision.HIGHEST,
                     preferred_element_type=jnp.float32)
        t = qv - r
        qst = r + t
        losses.append(jnp.sum(t * t))
        qsum = qsum + qst
        r = r - qst
        idx_ref[0, q] = jnp.broadcast_to(idxf.astype(jnp.int32), (blk, 8))
    q_ref[...] = qsum
    ri = jax.lax.broadcasted_iota(jnp.int32, (8, 128), 0)
    ci = jax.lax.broadcasted_iota(jnp.int32, (8, 128), 1)
    tile = jnp.zeros((8, 128), jnp.float32)
    for q in range(nq):
        tile = jnp.where((ri == q) & (ci == 0), losses[q], tile)
    loss_ref[0] = tile


def _vq(tokens, codebooks, cb2):
    n, c = tokens.shape
    nq, k, _ = codebooks.shape
    blk = _VQ_BLK
    grid = n // blk
    body = functools.partial(_vq_body, nq=nq, blk=blk, k=k)
    quant, idx, lossp = pl.pallas_call(
        body,
        grid=(grid,),
        in_specs=[
            pl.BlockSpec((blk, c), lambda i: (i, 0)),
            pl.BlockSpec(codebooks.shape, lambda i: (0, 0, 0)),
            pl.BlockSpec(cb2.shape, lambda i: (0, 0, 0)),
        ],
        out_specs=[
            pl.BlockSpec((blk, c), lambda i: (i, 0)),
            pl.BlockSpec((1, nq, blk, 8), lambda i: (i, 0, 0, 0)),
            pl.BlockSpec((1, 8, 128), lambda i: (i, 0, 0)),
        ],
        out_shape=[
            jax.ShapeDtypeStruct((n, c), jnp.float32),
            jax.ShapeDtypeStruct((grid, nq, blk, 8), jnp.int32),
            jax.ShapeDtypeStruct((grid, 8, 128), jnp.float32),
        ],
    )(tokens, codebooks, cb2)
    indices = idx[..., 0].transpose(1, 0, 2).reshape(nq, n)
    loss = lossp.sum(0)[:nq, 0] / (n * c)
    return quant, indices, loss


def _enc_conv(x, w, b, stride, pad):
    y = jax.lax.conv_general_dilated(
        x, w, (stride, stride), ((pad, pad), (pad, pad)),
        dimension_numbers=('NCHW', 'OIHW', 'NCHW'))
    return y + b[None, :, None, None]


def kernel(x, w1, b1, w2, b2, w3, b3, w4, b4, codebooks,
           dw0, db0, dtw1, dtb1, dtw2, dtb2):
    z = jax.nn.relu(_enc_conv(x, w1, b1, 2, 1))
    z = jax.nn.relu(_enc_conv(z, w2, b2, 2, 1))
    z = jax.nn.relu(_enc_conv(z, w3, b3, 1, 1))
    z = _enc_conv(z, w4, b4, 1, 1)
    bsz, c, h, _ = z.shape
    tokens = z.transpose(0, 2, 3, 1).reshape(bsz * h * h, c)

    cb2 = (codebooks ** 2).sum(-1)[:, None, :]
    quant, idx_flat, loss = _vq(tokens, codebooks, cb2)
    nq = codebooks.shape[0]
    indices = idx_flat.reshape(nq, bsz, h, h).transpose(1, 0, 2, 3)
    qmap_nhwc = quant.reshape(bsz, h, h, c)
    qmap = qmap_nhwc.transpose(0, 3, 1, 2)

    r = _conv_s1(qmap_nhwc, dw0, db0, "relu", 600)
    r = _conv_t(r, dtw1, dtb1, "relu")
    r = _conv_t(r, dtw2, dtb2, "tanh")
    recon = r.transpose(0, 3, 1, 2)
    return recon, indices, loss, qmap


# speedup vs baseline: 1.1479x; 1.1479x over previous
import functools

import jax
import jax.numpy as jnp
from jax.experimental import pallas as pl


def _taps_body(x_ref, w_ref, b_ref, o_ref, *, shifts, cols, blk, co, act, nsp):
    base = pl.program_id(0) * blk
    accs = [jnp.zeros((blk, co), jnp.float32) for _ in range(nsp)]
    for t, (sh, col) in enumerate(zip(shifts, cols)):
        al, rem = sh - sh % 8, sh % 8
        xs = x_ref[pl.ds(base + al, blk + 8), :][rem:rem + blk, :]
        accs[col] = accs[col] + jnp.dot(xs, w_ref[t],
                                        preferred_element_type=jnp.float32)
    y = jnp.concatenate(accs, axis=1) if nsp > 1 else accs[0]
    y = y + b_ref[...]
    if act == "relu":
        y = jnp.maximum(y, 0.0)
    elif act == "tanh":
        y = jnp.tanh(y)
    o_ref[...] = y


def _taps_matmul(x2, w, b, shifts, cols, n_valid, blk, act, front=0):
    t, k, co = w.shape
    ct = b.shape[1]
    nsp = ct // co
    grid = -(-n_valid // blk)
    n_out = grid * blk
    max_end = (grid - 1) * blk + max(shifts) + blk + 8
    x2p = jnp.pad(x2, ((front, max_end - front - x2.shape[0]), (0, 0)))
    body = functools.partial(_taps_body, shifts=tuple(shifts), cols=tuple(cols),
                             blk=blk, co=co, act=act, nsp=nsp)
    return pl.pallas_call(
        body,
        grid=(grid,),
        in_specs=[
            pl.BlockSpec(x2p.shape, lambda i: (0, 0)),
            pl.BlockSpec(w.shape, lambda i: (0, 0, 0)),
            pl.BlockSpec(b.shape, lambda i: (0, 0)),
        ],
        out_specs=pl.BlockSpec((blk, ct), lambda i: (i, 0)),
        out_shape=jax.ShapeDtypeStruct((n_out, ct), jnp.float32),
    )(x2p, w, b)


def _conv_s1(a, w, bias, act, blk):
    bsz, h, wd, cin = a.shape
    cout = w.shape[0]
    xp = jnp.pad(a, ((0, 0), (2, 2), (2, 2), (0, 0)))
    hp = h + 4
    x2 = xp.reshape(bsz * hp * hp, cin)
    front = hp + 1
    wt = w.transpose(2, 3, 1, 0)
    taps = [wt[kh, kw] for kh in range(3) for kw in range(3)]
    shifts = [front + (kh - 1) * hp + (kw - 1)
              for kh in range(3) for kw in range(3)]
    ws = jnp.stack(taps)
    n = bsz * hp * hp
    y = _taps_matmul(x2, ws, bias[None, :], shifts, [0] * 9, n, blk, act,
                     front=front)
    return y[:n].reshape(bsz, hp, hp, cout)[:, 2:2 + h, 2:2 + wd, :]


def _conv_t(a, w, bias, act):
    bsz, h, wd, cin = a.shape
    cout = w.shape[1]
    xp = jnp.pad(a, ((0, 0), (1, 1), (1, 1), (0, 0)))
    hp = h + 2
    x2 = xp.reshape(bsz * hp * hp, cin)
    taps, shifts, cols = [], [], []
    for r in (0, 1):
        for s in (0, 1):
            for di in (0, 1):
                for dj in (0, 1):
                    taps.append(w[:, :, 3 - 2 * di - r, 3 - 2 * dj - s])
                    shifts.append((r + di) * hp + (s + dj))
                    cols.append(2 * r + s)
    ws = jnp.stack(taps)
    btile = jnp.tile(bias[None, :], (1, 4))
    n = bsz * hp * hp
    y = _taps_matmul(x2, ws, btile, shifts, cols, n, 512, act)
    y = y[:n].reshape(bsz, hp, hp, 4, cout)[:, :h, :wd]
    y = y.reshape(bsz, h, wd, 2, 2, cout).transpose(0, 1, 3, 2, 4, 5)
    return y.reshape(bsz, 2 * h, 2 * wd, cout)


_VQ_BLK = 784


def _vq_body(x_ref, cbp_ref, cb2_ref, q_ref, idx_ref, loss_ref, *, nq, blk, k):
    r = x_ref[...]
    qsum = jnp.zeros_like(r)
    lanef = jax.lax.broadcasted_iota(jnp.int32, (blk, k), 1).astype(jnp.float32)
    losses = []
    for q in range(nq):
        p0 = cbp_ref[0, q]
        r2 = jnp.sum(r * r, axis=1, keepdims=True)
        mm = jax.lax.dot_general(r.astype(jnp.bfloat16), p0,
                                 (((1,), (1,)), ((), ())),
                                 preferred_element_type=jnp.float32)
        d = (r2 - 2.0 * mm) + cb2_ref[q]
        m = jnp.min(d, axis=1, keepdims=True)
        idxf = jnp.min(jnp.where(d <= m, lanef, jnp.float32(k)),
                       axis=1, keepdims=True)
        ohb = (lanef == idxf).astype(jnp.bfloat16)
        qv = jnp.dot(ohb, p0, preferred_element_type=jnp.float32)
        qv = qv + jnp.dot(ohb, cbp_ref[1, q], preferred_element_type=jnp.float32)
        qv = qv + jnp.dot(ohb, cbp_ref[2, q], preferred_element_type=jnp.float32)
        t = qv - r
        qst = r + t
        losses.append(jnp.sum(t * t))
        qsum = qsum + qst
        r = r - qst
        idx_ref[0, q] = jnp.broadcast_to(idxf.astype(jnp.int32), (blk, 8))
    q_ref[...] = qsum
    ri = jax.lax.broadcasted_iota(jnp.int32, (8, 128), 0)
    ci = jax.lax.broadcasted_iota(jnp.int32, (8, 128), 1)
    tile = jnp.zeros((8, 128), jnp.float32)
    for q in range(nq):
        tile = jnp.where((ri == q) & (ci == 0), losses[q], tile)
    loss_ref[0] = tile


def _vq(tokens, cbp, cb2):
    n, c = tokens.shape
    _, nq, k, _ = cbp.shape
    blk = _VQ_BLK
    grid = n // blk
    body = functools.partial(_vq_body, nq=nq, blk=blk, k=k)
    quant, idx, lossp = pl.pallas_call(
        body,
        grid=(grid,),
        in_specs=[
            pl.BlockSpec((blk, c), lambda i: (i, 0)),
            pl.BlockSpec(cbp.shape, lambda i: (0, 0, 0, 0)),
            pl.BlockSpec(cb2.shape, lambda i: (0, 0, 0)),
        ],
        out_specs=[
            pl.BlockSpec((blk, c), lambda i: (i, 0)),
            pl.BlockSpec((1, nq, blk, 8), lambda i: (i, 0, 0, 0)),
            pl.BlockSpec((1, 8, 128), lambda i: (i, 0, 0)),
        ],
        out_shape=[
            jax.ShapeDtypeStruct((n, c), jnp.float32),
            jax.ShapeDtypeStruct((grid, nq, blk, 8), jnp.int32),
            jax.ShapeDtypeStruct((grid, 8, 128), jnp.float32),
        ],
    )(tokens, cbp, cb2)
    indices = idx[..., 0].transpose(1, 0, 2).reshape(nq, n)
    loss = lossp.sum(0)[:nq, 0] / (n * c)
    return quant, indices, loss


def _enc_conv(x, w, b, stride, pad):
    y = jax.lax.conv_general_dilated(
        x, w, (stride, stride), ((pad, pad), (pad, pad)),
        dimension_numbers=('NCHW', 'OIHW', 'NCHW'))
    return y + b[None, :, None, None]


def kernel(x, w1, b1, w2, b2, w3, b3, w4, b4, codebooks,
           dw0, db0, dtw1, dtb1, dtw2, dtb2):
    z = jax.nn.relu(_enc_conv(x, w1, b1, 2, 1))
    z = jax.nn.relu(_enc_conv(z, w2, b2, 2, 1))
    z = jax.nn.relu(_enc_conv(z, w3, b3, 1, 1))
    z = _enc_conv(z, w4, b4, 1, 1)
    bsz, c, h, _ = z.shape
    tokens = z.transpose(0, 2, 3, 1).reshape(bsz * h * h, c)

    p0 = codebooks.astype(jnp.bfloat16)
    r1 = codebooks - p0.astype(jnp.float32)
    p1 = r1.astype(jnp.bfloat16)
    p2 = (r1 - p1.astype(jnp.float32)).astype(jnp.bfloat16)
    cbp = jnp.stack([p0, p1, p2])
    cb2 = (codebooks ** 2).sum(-1)[:, None, :]

    quant, idx_flat, loss = _vq(tokens, cbp, cb2)
    nq = codebooks.shape[0]
    indices = idx_flat.reshape(nq, bsz, h, h).transpose(1, 0, 2, 3)
    qmap_nhwc = quant.reshape(bsz, h, h, c)
    qmap = qmap_nhwc.transpose(0, 3, 1, 2)

    r = _conv_s1(qmap_nhwc, dw0, db0, "relu", 600)
    r = _conv_t(r, dtw1, dtb1, "relu")
    r = _conv_t(r, dtw2, dtb2, "tanh")
    recon = r.transpose(0, 3, 1, 2)
    return recon, indices, loss, qmap
